# baseline (speedup 1.0000x reference)
import functools

import jax
import jax.numpy as jnp
import numpy as np
from jax import lax
from jax.experimental import pallas as pl
from jax.experimental.pallas import tpu as pltpu

E = 32
G = 4
TOPK = 700
P = 3136
H = 56
KPAD = 704
WLANES = 512

_INTERPRET = False


def _heads_body(x_ref, wb_ref, bb_ref, wo_ref, bo_ref, we_ref, be_ref,
                ww_ref, bw_ref, obj_ref, enc_ref, wmap_ref):
    feat = jnp.maximum(
        jnp.dot(x_ref[...], wb_ref[...], preferred_element_type=jnp.float32)
        + bb_ref[...], 0.0)
    obj_ref[...] = (
        jnp.dot(feat, wo_ref[...], preferred_element_type=jnp.float32)
        + bo_ref[...])
    enc_ref[...] = (
        jnp.dot(feat, we_ref[...], preferred_element_type=jnp.float32)
        + be_ref[...])
    wmap_ref[...] = (
        jnp.dot(feat, ww_ref[...], preferred_element_type=jnp.float32)
        + bw_ref[...])


def _topk_gather_body(vrow_ref, vcol_ref, wmap_ref, wsel_ref, scores_ref):
    s_row = jax.nn.sigmoid(vrow_ref[...])
    s_col = jax.nn.sigmoid(vcol_ref[...])
    rank = jnp.zeros((1, P), jnp.int32)
    jblk = 448
    for b in range(P // jblk):
        sj = s_col[b * jblk:(b + 1) * jblk, :]
        jidx = b * jblk + jax.lax.broadcasted_iota(jnp.int32, (jblk, P), 0)
        iidx = jax.lax.broadcasted_iota(jnp.int32, (jblk, P), 1)
        gt = sj > s_row
        eq = (sj == s_row) & (jidx < iidx)
        rank = rank + jnp.sum((gt | eq).astype(jnp.int32), axis=0,
                              keepdims=True)
    kblk = 176
    for b in range(KPAD // kblk):
        kidx = b * kblk + jax.lax.broadcasted_iota(jnp.int32, (kblk, P), 0)
        oneh = (kidx == rank).astype(jnp.float32)
        wsel_ref[b * kblk:(b + 1) * kblk, :] = jnp.dot(
            oneh, wmap_ref[...], preferred_element_type=jnp.float32)
        scores_ref[b * kblk:(b + 1) * kblk, :] = jnp.sum(
            oneh * s_row, axis=1, keepdims=True)


_LOG2E = 1.4426950408889634


def _decup_body(wsel_ref, enc_ref, ut_ref, u_ref, out_ref, scratch, sems,
                kb, nsteps):
    i = pl.program_id(0)
    slot = lax.rem(i, 2)

    @pl.when(i >= 2)
    def _wait_slot():
        pltpu.make_async_copy(
            scratch.at[slot], out_ref.at[0, pl.ds((i - 2) * kb, kb)],
            sems.at[slot]).wait()

    acc = None
    for g in range(G):
        wg = wsel_ref[:, 128 * g:128 * g + E + 1]
        z = jnp.dot(wg, enc_ref[...],
                    preferred_element_type=jnp.float32)
        q = 1.0 + jnp.exp2(z * (-_LOG2E))
        acc = q if acc is None else acc * q
    m = 1.0 / acc
    a = m.reshape(kb * H, 128)
    x1 = jnp.dot(a, ut_ref[...],
                 preferred_element_type=jnp.float32)
    for k in range(kb):
        scratch[slot, k] = jnp.dot(u_ref[...], x1[k * H:(k + 1) * H, :],
                                   preferred_element_type=jnp.float32)

    @pl.when(i < nsteps - 1)
    def _start_full():
        pltpu.make_async_copy(
            scratch.at[slot], out_ref.at[0, pl.ds(i * kb, kb)],
            sems.at[slot]).start()

    @pl.when(i == nsteps - 1)
    def _last():
        tail = TOPK - (nsteps - 1) * kb
        pltpu.make_async_copy(
            scratch.at[0, pl.ds(0, tail)],
            out_ref.at[0, pl.ds((nsteps - 1) * kb, tail)],
            sems.at[0]).start()
        pltpu.make_async_copy(
            scratch.at[1], out_ref.at[0, pl.ds((nsteps - 2) * kb, kb)],
            sems.at[1]).wait()
        pltpu.make_async_copy(
            scratch.at[0, pl.ds(0, tail)],
            out_ref.at[0, pl.ds((nsteps - 1) * kb, tail)],
            sems.at[0]).wait()


def _upsample_matrix():
    o = np.arange(4 * H)
    pos = (o + 0.5) / 4.0 - 0.5
    lo = np.floor(pos).astype(np.int64)
    w = (pos - lo).astype(np.float32)
    u = np.zeros((4 * H, H), np.float32)
    for i in range(4 * H):
        l = min(max(int(lo[i]), 0), H - 1)
        h = min(max(int(lo[i]) + 1, 0), H - 1)
        u[i, l] += 1.0 - w[i]
        u[i, h] += w[i]
    return u


def kernel(image, Wb, bb, Wo, bo, We, be, Ww, bw):
    f32 = jnp.float32
    x = image.reshape(3, H, 4, H, 4).transpose(1, 3, 0, 2, 4).reshape(P, 48)
    wb_t = Wb.reshape(96, 48).T
    bb2 = bb.reshape(1, 96)
    wo_t = jnp.zeros((96, 128), f32).at[:, 0].set(Wo[0])
    bo2 = jnp.zeros((1, 128), f32).at[0, 0].set(bo[0])
    we_t = We.T
    be2 = be.reshape(1, E)
    lane = (128 * (np.arange((E + 1) * G) // (E + 1))
            + np.arange((E + 1) * G) % (E + 1))
    ww_t = jnp.zeros((96, WLANES), f32).at[:, lane].set(Ww.T)
    bw2 = jnp.zeros((1, WLANES), f32).at[0, lane].set(bw)

    obj_full, enc, wmap = pl.pallas_call(
        _heads_body,
        out_shape=(
            jax.ShapeDtypeStruct((P, 128), f32),
            jax.ShapeDtypeStruct((P, E), f32),
            jax.ShapeDtypeStruct((P, WLANES), f32),
        ),
        interpret=_INTERPRET,
    )(x, wb_t, bb2, wo_t, bo2, we_t, be2, ww_t, bw2)

    obj_col = obj_full[:, :1]
    obj_row = obj_col.reshape(1, P)

    wsel, scores = pl.pallas_call(
        _topk_gather_body,
        out_shape=(
            jax.ShapeDtypeStruct((KPAD, WLANES), f32),
            jax.ShapeDtypeStruct((KPAD, 1), f32),
        ),
        interpret=_INTERPRET,
    )(obj_row, obj_col, wmap)

    enc_t = enc.T
    enc_aug = jnp.zeros((E + 1, H, 128), f32)
    enc_aug = enc_aug.at[:E, :, :H].set(enc_t.reshape(E, H, H))
    enc_aug = enc_aug.at[E, :, :H].set(1.0)
    enc_aug = enc_aug.reshape(E + 1, H * 128)
    u = jnp.asarray(_upsample_matrix())
    ut128 = np.zeros((128, 4 * H), np.float32)
    ut128[:H, :] = _upsample_matrix().T
    ut = jnp.asarray(ut128)
    kb = 56
    nsteps = 13
    masks = pl.pallas_call(
        functools.partial(_decup_body, kb=kb, nsteps=nsteps),
        grid=(nsteps,),
        in_specs=[
            pl.BlockSpec((kb, WLANES), lambda i: (i, 0)),
            pl.BlockSpec((E + 1, H * 128), lambda i: (0, 0)),
            pl.BlockSpec((128, 4 * H), lambda i: (0, 0)),
            pl.BlockSpec((4 * H, H), lambda i: (0, 0)),
        ],
        out_specs=pl.BlockSpec(memory_space=pl.ANY),
        out_shape=jax.ShapeDtypeStruct((1, TOPK, 4 * H, 4 * H), f32),
        scratch_shapes=[
            pltpu.VMEM((2, kb, 4 * H, 4 * H), f32),
            pltpu.SemaphoreType.DMA((2,)),
        ],
        interpret=_INTERPRET,
    )(wsel, enc_aug, ut, u)

    obj_logits = obj_col.reshape(1, 1, H, H)
    return obj_logits, masks, scores[:TOPK, 0].reshape(1, TOPK)
